# Initial kernel scaffold; baseline (speedup 1.0000x reference)
import jax
import jax.numpy as jnp
from jax.experimental import pallas as pl


def kernel(emb_sentences, att_sentences, W, b):
    raise NotImplementedError("write your pallas kernel here")


# speedup vs baseline: 1.2781x; 1.2781x over previous
import jax
import jax.numpy as jnp
from jax.experimental import pallas as pl


def _fused_kernel(mask_ref, w_ref, b_ref, emb_ref, emb_out_ref, logits_ref):
    x = emb_ref[0, 0]
    emb_out_ref[0, 0] = x
    y = jnp.dot(x, w_ref[0], preferred_element_type=jnp.float32)
    y = y + b_ref[0]
    logits_ref[0, 0] = y + mask_ref[0]


@jax.jit
def _run(emb_sentences, mask, W, b3):
    B, L, S, D = emb_sentences.shape
    C = W.shape[-1]
    BS = 512
    grid = (B, L, S // BS)

    emb_out, logits = pl.pallas_call(
        _fused_kernel,
        grid=grid,
        in_specs=[
            pl.BlockSpec((1, BS, 1), lambda bi, li, si: (bi, si, 0)),
            pl.BlockSpec((1, D, C), lambda bi, li, si: (li, 0, 0)),
            pl.BlockSpec((1, 1, C), lambda bi, li, si: (li, 0, 0)),
            pl.BlockSpec((1, 1, BS, D), lambda bi, li, si: (bi, li, si, 0)),
        ],
        out_specs=[
            pl.BlockSpec((1, 1, BS, D), lambda bi, li, si: (bi, li, si, 0)),
            pl.BlockSpec((1, 1, BS, C), lambda bi, li, si: (bi, li, si, 0)),
        ],
        out_shape=[
            jax.ShapeDtypeStruct((B, L, S, D), jnp.float32),
            jax.ShapeDtypeStruct((B, L, S, C), jnp.float32),
        ],
    )(mask, W, b3, emb_sentences)
    return emb_out, logits


def kernel(emb_sentences, att_sentences, W, b):
    B, L, S, D = emb_sentences.shape
    mask = jnp.where(att_sentences, 0.0, -jnp.inf).astype(jnp.float32)
    mask = mask.reshape(B, S, 1)
    b3 = b.reshape(b.shape[0], 1, b.shape[1])
    emb_out, logits = _run(emb_sentences, mask, W, b3)
    return emb_out, att_sentences, logits


# speedup vs baseline: 1.2789x; 1.0006x over previous
import jax
import jax.numpy as jnp
from jax.experimental import pallas as pl
from jax.experimental.pallas import tpu as pltpu


def _fused_kernel(mask_ref, w_ref, b_ref, emb_ref, emb_out_ref, logits_ref):
    x = emb_ref[0, 0]
    emb_out_ref[0, 0] = x
    y = jnp.dot(x, w_ref[0], preferred_element_type=jnp.float32)
    y = y + b_ref[0]
    logits_ref[0, 0] = y + mask_ref[0]


@jax.jit
def _run(emb_sentences, mask, W, b3):
    B, L, S, D = emb_sentences.shape
    C = W.shape[-1]
    BS = 512
    grid = (B, L, S // BS)

    emb_out, logits = pl.pallas_call(
        _fused_kernel,
        grid=grid,
        in_specs=[
            pl.BlockSpec((1, BS, 1), lambda bi, li, si: (bi, si, 0)),
            pl.BlockSpec((1, D, C), lambda bi, li, si: (li, 0, 0)),
            pl.BlockSpec((1, 1, C), lambda bi, li, si: (li, 0, 0)),
            pl.BlockSpec((1, 1, BS, D), lambda bi, li, si: (bi, li, si, 0)),
        ],
        out_specs=[
            pl.BlockSpec((1, 1, BS, D), lambda bi, li, si: (bi, li, si, 0)),
            pl.BlockSpec((1, 1, BS, C), lambda bi, li, si: (bi, li, si, 0)),
        ],
        out_shape=[
            jax.ShapeDtypeStruct((B, L, S, D), jnp.float32),
            jax.ShapeDtypeStruct((B, L, S, C), jnp.float32),
        ],
        compiler_params=pltpu.CompilerParams(
            dimension_semantics=("parallel", "parallel", "parallel"),
        ),
    )(mask, W, b3, emb_sentences)
    return emb_out, logits


def kernel(emb_sentences, att_sentences, W, b):
    B, L, S, D = emb_sentences.shape
    mask = jnp.where(att_sentences, 0.0, -jnp.inf).astype(jnp.float32)
    mask = mask.reshape(B, S, 1)
    b3 = b.reshape(b.shape[0], 1, b.shape[1])
    emb_out, logits = _run(emb_sentences, mask, W, b3)
    return emb_out, att_sentences, logits


# speedup vs baseline: 1.4906x; 1.1655x over previous
import jax
import jax.numpy as jnp
from jax.experimental import pallas as pl
from jax.experimental.pallas import tpu as pltpu


def _fused_kernel(mask_ref, w_ref, b_ref, emb_ref, emb_out_ref, logits_ref):
    x = emb_ref[0, 0]
    emb_out_ref[0, 0] = x
    y = jnp.dot(x, w_ref[0], preferred_element_type=jnp.float32)
    y = y + b_ref[0]
    logits_ref[0, 0] = y + mask_ref[0]


@jax.jit
def _run(emb_sentences, mask, W, b3):
    B, L, S, D = emb_sentences.shape
    C = W.shape[-1]
    BS = 1024
    grid = (B, L, S // BS)

    emb_out, logits = pl.pallas_call(
        _fused_kernel,
        grid=grid,
        in_specs=[
            pl.BlockSpec((1, BS, 1), lambda bi, li, si: (bi, si, 0)),
            pl.BlockSpec((1, D, C), lambda bi, li, si: (li, 0, 0)),
            pl.BlockSpec((1, 1, C), lambda bi, li, si: (li, 0, 0)),
            pl.BlockSpec((1, 1, BS, D), lambda bi, li, si: (bi, li, si, 0)),
        ],
        out_specs=[
            pl.BlockSpec((1, 1, BS, D), lambda bi, li, si: (bi, li, si, 0)),
            pl.BlockSpec((1, 1, BS, C), lambda bi, li, si: (bi, li, si, 0)),
        ],
        out_shape=[
            jax.ShapeDtypeStruct((B, L, S, D), jnp.float32),
            jax.ShapeDtypeStruct((B, L, S, C), jnp.float32),
        ],
        compiler_params=pltpu.CompilerParams(
            dimension_semantics=("parallel", "parallel", "parallel"),
        ),
    )(mask, W, b3, emb_sentences)
    return emb_out, logits


def kernel(emb_sentences, att_sentences, W, b):
    B, L, S, D = emb_sentences.shape
    mask = jnp.where(att_sentences, 0.0, -jnp.inf).astype(jnp.float32)
    mask = mask.reshape(B, S, 1)
    b3 = b.reshape(b.shape[0], 1, b.shape[1])
    emb_out, logits = _run(emb_sentences, mask, W, b3)
    return emb_out, att_sentences, logits


# speedup vs baseline: 1.5916x; 1.0678x over previous
import jax
import jax.numpy as jnp
from jax.experimental import pallas as pl
from jax.experimental.pallas import tpu as pltpu


def _fused_kernel(mask_ref, w_ref, b_ref, emb_ref, emb_out_ref, logits_ref):
    x = emb_ref[0, 0]
    emb_out_ref[0, 0] = x
    y = jnp.dot(x, w_ref[0], preferred_element_type=jnp.float32)
    y = y + b_ref[0]
    logits_ref[0, 0] = y + mask_ref[0]


@jax.jit
def _run(emb_sentences, mask, W, b3):
    B, L, S, D = emb_sentences.shape
    C = W.shape[-1]
    BS = 2048
    grid = (B, L, S // BS)

    emb_out, logits = pl.pallas_call(
        _fused_kernel,
        grid=grid,
        in_specs=[
            pl.BlockSpec((1, BS, 1), lambda bi, li, si: (bi, si, 0)),
            pl.BlockSpec((1, D, C), lambda bi, li, si: (li, 0, 0)),
            pl.BlockSpec((1, 1, C), lambda bi, li, si: (li, 0, 0)),
            pl.BlockSpec((1, 1, BS, D), lambda bi, li, si: (bi, li, si, 0)),
        ],
        out_specs=[
            pl.BlockSpec((1, 1, BS, D), lambda bi, li, si: (bi, li, si, 0)),
            pl.BlockSpec((1, 1, BS, C), lambda bi, li, si: (bi, li, si, 0)),
        ],
        out_shape=[
            jax.ShapeDtypeStruct((B, L, S, D), jnp.float32),
            jax.ShapeDtypeStruct((B, L, S, C), jnp.float32),
        ],
        compiler_params=pltpu.CompilerParams(
            dimension_semantics=("parallel", "parallel", "parallel"),
        ),
    )(mask, W, b3, emb_sentences)
    return emb_out, logits


def kernel(emb_sentences, att_sentences, W, b):
    B, L, S, D = emb_sentences.shape
    mask = jnp.where(att_sentences, 0.0, -jnp.inf).astype(jnp.float32)
    mask = mask.reshape(B, S, 1)
    b3 = b.reshape(b.shape[0], 1, b.shape[1])
    emb_out, logits = _run(emb_sentences, mask, W, b3)
    return emb_out, att_sentences, logits
